# baseline (speedup 1.0000x reference)
import functools

import jax
import jax.numpy as jnp
from jax import lax
from jax.experimental import pallas as pl
from jax.experimental.pallas import tpu as pltpu
from jax.experimental.pallas import tpu_sc as plsc

_VOCAB = 1000
_NC = 2
_NS = 16
_NW = _NC * _NS
_CHUNK = 32
_NBUF = 4
_LAG = 2


@functools.lru_cache(maxsize=None)
def _make_gather(bt: int, vocab: int):
    per_w = bt // _NW
    assert per_w * _NW == bt and per_w % _CHUNK == 0 and _CHUNK % 8 == 0
    nchunk = per_w // _CHUNK
    main = (nchunk - _LAG) // _NBUF * _NBUF
    mesh = plsc.VectorSubcoreMesh(core_axis_name="c", subcore_axis_name="s")

    @functools.partial(
        pl.kernel,
        mesh=mesh,
        compiler_params=pltpu.CompilerParams(use_tc_tiling_on_sc=False),
        out_type=jax.ShapeDtypeStruct((bt, vocab), jnp.float32),
        scratch_types=[
            pltpu.VMEM((per_w,), jnp.int32),
            [pltpu.VMEM((_CHUNK, vocab), jnp.float32) for _ in range(_NBUF)],
            [pltpu.SemaphoreType.DMA for _ in range(_NBUF)],
            [pltpu.SemaphoreType.DMA for _ in range(_NBUF)],
        ],
    )
    def gather(idx_hbm, table_hbm, out_hbm, idx_v, rows, gsem, wsem):
        wid = lax.axis_index("s") * _NC + lax.axis_index("c")
        base = pl.multiple_of(wid * per_w, 8)
        pltpu.sync_copy(idx_hbm.at[pl.ds(base, per_w)], idx_v)

        def issue_gather(c, b):
            off = pl.multiple_of(c * _CHUNK, 8)
            pltpu.async_copy(table_hbm.at[idx_v.at[pl.ds(off, _CHUNK)]],
                             rows[b], gsem[b])

        def wait_gather(c, b):
            off = pl.multiple_of(c * _CHUNK, 8)
            pltpu.make_async_copy(table_hbm.at[idx_v.at[pl.ds(off, _CHUNK)]],
                                  rows[b], gsem[b]).wait()

        def issue_write(c, b):
            off = pl.multiple_of(base + c * _CHUNK, 8)
            pltpu.async_copy(rows[b], out_hbm.at[pl.ds(off, _CHUNK)], wsem[b])

        def wait_write(c, b):
            off = pl.multiple_of(base + c * _CHUNK, 8)
            pltpu.make_async_copy(rows[b], out_hbm.at[pl.ds(off, _CHUNK)],
                                  wsem[b]).wait()

        @pl.loop(0, main, step=_NBUF)
        def _body(c0):
            for b in range(_NBUF):
                c = c0 + b

                @pl.when(c >= _NBUF)
                def _():
                    wait_write(c - _NBUF, b)

                issue_gather(c, b)
                cp = c - _LAG

                @pl.when(cp >= 0)
                def _():
                    wait_gather(cp, (b - _LAG) % _NBUF)
                    issue_write(cp, (b - _LAG) % _NBUF)

        for c in range(main, nchunk):
            b = c % _NBUF
            if c >= _NBUF:
                wait_write(c - _NBUF, b)
            issue_gather(c, b)
            cp = c - _LAG
            if cp >= 0:
                wait_gather(cp, cp % _NBUF)
                issue_write(cp, cp % _NBUF)
        for c in range(max(0, nchunk - _LAG), nchunk):
            wait_gather(c, c % _NBUF)
            issue_write(c, c % _NBUF)
        for c in range(max(0, nchunk - _NBUF), nchunk):
            wait_write(c, c % _NBUF)

    return gather


def kernel(idx, table):
    b, t = idx.shape
    flat = idx.reshape(-1).astype(jnp.int32)
    out = _make_gather(b * t, table.shape[1])(flat, table)
    return out.reshape(b, t, table.shape[1])


# speedup vs baseline: 1.0042x; 1.0042x over previous
import functools

import jax
import jax.numpy as jnp
from jax import lax
from jax.experimental import pallas as pl
from jax.experimental.pallas import tpu as pltpu
from jax.experimental.pallas import tpu_sc as plsc

_NC = 2
_NS = 16
_NW = _NC * _NS
_NBUF = 2


@functools.lru_cache(maxsize=None)
def _make_gather(b: int, t: int, vocab: int):
    per_w = b // _NW
    assert per_w * _NW == b and per_w % _NBUF == 0
    mesh = plsc.VectorSubcoreMesh(core_axis_name="c", subcore_axis_name="s")

    @functools.partial(
        pl.kernel,
        mesh=mesh,
        compiler_params=pltpu.CompilerParams(use_tc_tiling_on_sc=False),
        out_type=jax.ShapeDtypeStruct((b, t, vocab), jnp.float32),
        scratch_types=[
            pltpu.VMEM((per_w, t), jnp.int32),
            [pltpu.VMEM((t, vocab), jnp.float32) for _ in range(_NBUF)],
            [pltpu.SemaphoreType.DMA for _ in range(_NBUF)],
            [pltpu.SemaphoreType.DMA for _ in range(_NBUF)],
        ],
    )
    def gather(idx_hbm, table_hbm, out_hbm, idx_v, rows, gsem, wsem):
        wid = lax.axis_index("s") * _NC + lax.axis_index("c")
        b0 = pl.multiple_of(wid * per_w, 8)
        pltpu.sync_copy(idx_hbm.at[pl.ds(b0, per_w)], idx_v)

        def issue_gather(c, buf):
            pltpu.async_copy(table_hbm.at[idx_v.at[c]], rows[buf], gsem[buf])

        def wait_gather(c, buf):
            pltpu.make_async_copy(table_hbm.at[idx_v.at[c]], rows[buf],
                                  gsem[buf]).wait()

        def issue_write(c, buf):
            pltpu.async_copy(rows[buf], out_hbm.at[b0 + c], wsem[buf])

        def wait_write(c, buf):
            pltpu.make_async_copy(rows[buf], out_hbm.at[b0 + c],
                                  wsem[buf]).wait()

        @pl.loop(0, per_w, step=_NBUF)
        def _body(c0):
            for bb in range(_NBUF):
                c = c0 + bb

                @pl.when(c >= _NBUF)
                def _():
                    wait_write(c - _NBUF, bb)

                issue_gather(c, bb)

                @pl.when(c >= 1)
                def _():
                    wait_gather(c - 1, (bb - 1) % _NBUF)
                    issue_write(c - 1, (bb - 1) % _NBUF)

        last = per_w - 1
        wait_gather(last, last % _NBUF)
        issue_write(last, last % _NBUF)
        for c in range(per_w - _NBUF, per_w):
            wait_write(c, c % _NBUF)

    return gather


def kernel(idx, table):
    b, t = idx.shape
    return _make_gather(b, t, table.shape[1])(idx.astype(jnp.int32), table)
